# baseline (speedup 1.0000x reference)
import functools

import jax
import jax.numpy as jnp
from jax import lax
from jax.experimental import pallas as pl
from jax.experimental.pallas import tpu as pltpu

BATCH = 16384
EMBED = 128
ROWS_PER_BLOCK = 2048


def _score_body(h_ref, t_ref, r_ref, o_ref):
    a = jnp.abs(h_ref[...] + r_ref[...] - t_ref[...])
    w = EMBED
    while w > 1:
        w //= 2
        a = a[:, :w] + a[:, w:]
    o_ref[...] = -a[:, 0]


def _tc_score(ph, pt, pr):
    grid = BATCH // ROWS_PER_BLOCK
    spec = pl.BlockSpec((ROWS_PER_BLOCK, EMBED), lambda i: (i, 0))
    return pl.pallas_call(
        _score_body,
        grid=(grid,),
        in_specs=[spec, spec, spec],
        out_specs=pl.BlockSpec((ROWS_PER_BLOCK,), lambda i: (i,)),
        out_shape=jax.ShapeDtypeStruct((BATCH,), jnp.float32),
    )(ph, pt, pr)


def kernel(h_idx, t_idx, r_idx, ent_table, rel_table):
    ph = jnp.take(ent_table, h_idx, axis=0)
    pt = jnp.take(ent_table, t_idx, axis=0)
    pr = jnp.take(rel_table, r_idx, axis=0)
    p_score = _tc_score(ph, pt, pr)
    ranked = jnp.argsort(-p_score)
    return (p_score, ranked)
